# Initial kernel scaffold; baseline (speedup 1.0000x reference)
import jax
import jax.numpy as jnp
from jax.experimental import pallas as pl


def kernel(x, x_mark, edge_index, weights, lin_W, lin_b, emb_table, gcn_W, gcn_b):
    raise NotImplementedError("write your pallas kernel here")


# speedup vs baseline: 7.6349x; 7.6349x over previous
import math

import jax
import jax.numpy as jnp
import numpy as np
from jax import lax
from jax.experimental import pallas as pl
from jax.experimental.pallas import tpu as pltpu
from jax.experimental.pallas import tpu_sc as plsc

B = 4
N = 10000
NP = 10240
E = 320000
T = 12
D = 128
NC = 2
NS = 16
NW = NC * NS
EPW = E // NW
P2C = 1
NW2 = P2C * NS
EPW2 = E // NW2
CHUNK = 80
SEG = 2000
NSEG = EPW2 // SEG
CPS = SEG // CHUNK

_pos = np.arange(T, dtype=np.float64)[:, None]
_div = np.exp(np.arange(0, D, 2, dtype=np.float64) * (-(math.log(10000.0) / D)))
_pe = np.zeros((T, D), dtype=np.float32)
_pe[:, 0::2] = np.sin(_pos * _div)
_pe[:, 1::2] = np.cos(_pos * _div)


def _deg_kernel(col_hbm, ew_hbm, out_hbm, col_v, ew_v, deg_v):
    cid = lax.axis_index("c")
    sid = lax.axis_index("s")
    wid = sid * NC + cid
    pltpu.sync_copy(col_hbm.at[wid], col_v)
    pltpu.sync_copy(ew_hbm.at[wid], ew_v)
    zero16 = jnp.zeros((16,), jnp.float32)

    def zloop(i, carry):
        deg_v[pl.ds(i * 16, 16)] = zero16
        return carry

    lax.fori_loop(0, NP // 16, zloop, 0)

    def eloop(i, carry):
        s = pl.ds(i * 16, 16)
        plsc.addupdate_scatter(deg_v, [col_v[s]], ew_v[s])
        return carry

    lax.fori_loop(0, EPW // 16, eloop, 0)
    pltpu.sync_copy(deg_v, out_hbm.at[wid])


def _run_deg(col2, ew2):
    f = pl.kernel(
        _deg_kernel,
        out_type=jax.ShapeDtypeStruct((NW, NP), jnp.float32),
        mesh=plsc.VectorSubcoreMesh(core_axis_name="c", subcore_axis_name="s"),
        compiler_params=pltpu.CompilerParams(needs_layout_passes=False),
        scratch_types=[
            pltpu.VMEM((EPW,), jnp.int32),
            pltpu.VMEM((EPW,), jnp.float32),
            pltpu.VMEM((NP,), jnp.float32),
        ],
    )
    return f(col2, ew2)


def _tc1_kernel(emb_ref, wt_ref, degT_ref, xlin_ref, dinv_ref):
    xlin_ref[...] = jnp.dot(emb_ref[...], wt_ref[...],
                            preferred_element_type=jnp.float32)
    deg = jnp.sum(degT_ref[...], axis=1, keepdims=True) + 1.0
    dinv_ref[...] = lax.rsqrt(deg)


def _run_tc1(emb_pad, gcn_Wt, degT):
    nb = 1280
    grid = (NP // nb,)
    return pl.pallas_call(
        _tc1_kernel,
        grid=grid,
        in_specs=[
            pl.BlockSpec((nb, D), lambda i: (i, 0)),
            pl.BlockSpec((D, D), lambda i: (0, 0)),
            pl.BlockSpec((nb, NW), lambda i: (i, 0)),
        ],
        out_specs=[
            pl.BlockSpec((nb, D), lambda i: (i, 0)),
            pl.BlockSpec((nb, 1), lambda i: (i, 0)),
        ],
        out_shape=[
            jax.ShapeDtypeStruct((NP, D), jnp.float32),
            jax.ShapeDtypeStruct((NP, 1), jnp.float32),
        ],
    )(emb_pad, gcn_Wt, degT)


def _edge_kernel(xlin_hbm, dinv_hbm, rowf_hbm, colf_hbm, col3_hbm, ew_hbm,
                 out_hbm, dinv_v, rowf_v, colf_v, ew_v, col2d_v, norm_v,
                 rows_v, zbuf_v, acc, gsem):
    sid = lax.axis_index("s")
    wid = sid
    pltpu.sync_copy(dinv_hbm, dinv_v)

    zero16 = jnp.zeros((16,), jnp.float32)

    def zloop(i, carry):
        for j in range(8):
            zbuf_v[i, pl.ds(j * 16, 16)] = zero16
        return carry

    lax.fori_loop(0, 32, zloop, 0)
    rows_per_sub = NP // NS
    for k in range(rows_per_sub // 32):
        pltpu.sync_copy(zbuf_v, acc.at[pl.ds(sid * rows_per_sub + k * 32, 32)])
    plsc.subcore_barrier()

    def sloop(g, carry):
        pltpu.sync_copy(rowf_hbm.at[wid, g], rowf_v)
        pltpu.sync_copy(colf_hbm.at[wid, g], colf_v)
        pltpu.sync_copy(col3_hbm.at[wid, g], col2d_v)
        pltpu.sync_copy(ew_hbm.at[wid, g], ew_v)

        def nloop(i, carry1):
            s = pl.ds(i * 16, 16)
            dr = plsc.load_gather(dinv_v, [rowf_v[s]])
            dc = plsc.load_gather(dinv_v, [colf_v[s]])
            norm_v[s] = dr * ew_v[s] * dc
            return carry1

        lax.fori_loop(0, SEG // 16, nloop, 0)

        def cloop(c, carry1):
            pltpu.async_copy(
                xlin_hbm.at[rowf_v.at[pl.ds(c * CHUNK, CHUNK)]], rows_v, gsem
            ).wait()

            def eloop(e, carry2):
                splat = plsc.load_gather(
                    norm_v, [jnp.full((16,), c * CHUNK + e, jnp.int32)])
                for j in range(8):
                    sl = pl.ds(j * 16, 16)
                    rows_v[e, sl] = rows_v[e, sl] * splat
                return carry2

            lax.fori_loop(0, CHUNK, eloop, 0)
            pltpu.sync_copy(rows_v, acc.at[col2d_v.at[c]], add=True)
            return carry1

        lax.fori_loop(0, CPS, cloop, 0)
        return carry

    lax.fori_loop(0, NSEG, sloop, 0)
    plsc.subcore_barrier()
    pltpu.sync_copy(acc.at[pl.ds(sid * rows_per_sub, rows_per_sub)],
                    out_hbm.at[0, pl.ds(sid * rows_per_sub, rows_per_sub)])


def _run_edges(xlin, dinv_flat, rowf2, colf2, col3, ew2):
    f = pl.kernel(
        _edge_kernel,
        out_type=jax.ShapeDtypeStruct((P2C, NP, D), jnp.float32),
        mesh=plsc.VectorSubcoreMesh(core_axis_name="c", subcore_axis_name="s",
                                    num_cores=P2C),
        compiler_params=pltpu.CompilerParams(needs_layout_passes=False),
        scratch_types=[
            pltpu.VMEM((NP,), jnp.float32),
            pltpu.VMEM((SEG,), jnp.int32),
            pltpu.VMEM((SEG,), jnp.int32),
            pltpu.VMEM((SEG,), jnp.float32),
            pltpu.VMEM((CPS, CHUNK), jnp.int32),
            pltpu.VMEM((SEG,), jnp.float32),
            pltpu.VMEM((CHUNK, D), jnp.float32),
            pltpu.VMEM((32, D), jnp.float32),
            pltpu.VMEM_SHARED((NP, D), jnp.float32),
            pltpu.SemaphoreType.DMA,
        ],
    )
    return f(xlin, dinv_flat, rowf2, colf2, col3, ew2)


def _tc2_kernel(x_ref, parts_ref, xlin_ref, dinv_ref, pe_ref, w_ref, lb_ref,
                gb_ref, out_ref):
    dinv = dinv_ref[...]
    edge_sum = parts_ref[0]
    for c in range(1, P2C):
        edge_sum = edge_sum + parts_ref[c]
    embed = edge_sum + xlin_ref[...] * (dinv * dinv) + gb_ref[...]
    base = embed[:, None, :] + (pe_ref[...] + lb_ref[...])[None, :, :]
    xb = x_ref[0][:, :, None]
    out_ref[0] = xb * w_ref[...][None, :, :] + base


def _run_tc2(x3, parts, xlin, dinv, pe, w_row, lb_row, gb_row):
    nb = 400
    grid = (N // nb, B)
    return pl.pallas_call(
        _tc2_kernel,
        grid=grid,
        in_specs=[
            pl.BlockSpec((1, nb, T), lambda n, b: (b, n, 0)),
            pl.BlockSpec((P2C, nb, D), lambda n, b: (0, n, 0)),
            pl.BlockSpec((nb, D), lambda n, b: (n, 0)),
            pl.BlockSpec((nb, 1), lambda n, b: (n, 0)),
            pl.BlockSpec((T, D), lambda n, b: (0, 0)),
            pl.BlockSpec((1, D), lambda n, b: (0, 0)),
            pl.BlockSpec((1, D), lambda n, b: (0, 0)),
            pl.BlockSpec((1, D), lambda n, b: (0, 0)),
        ],
        out_specs=pl.BlockSpec((1, nb, T, D), lambda n, b: (b, n, 0, 0)),
        out_shape=jax.ShapeDtypeStruct((B, N, T, D), jnp.float32),
    )(x3, parts, xlin, dinv, pe, w_row, lb_row, gb_row)


def kernel(x, x_mark, edge_index, weights, lin_W, lin_b, emb_table, gcn_W,
           gcn_b):
    del x_mark
    row = edge_index[0]
    col = edge_index[1]
    colf1 = col.reshape(NW, EPW)
    ew1 = weights.reshape(NW, EPW)
    rowf2 = row.reshape(NW2, NSEG, SEG)
    colf2 = col.reshape(NW2, NSEG, SEG)
    col3 = col.reshape(NW2, NSEG, CPS, CHUNK)
    ew2 = weights.reshape(NW2, NSEG, SEG)

    emb_pad = jnp.pad(emb_table, ((0, NP - N), (0, 0)))
    gcn_Wt = gcn_W.T

    deg_parts = _run_deg(colf1, ew1)
    degT = deg_parts.T
    xlin, dinv = _run_tc1(emb_pad, gcn_Wt, degT)
    parts = _run_edges(xlin, dinv.reshape(NP), rowf2, colf2, col3, ew2)

    x3 = x[..., 0]
    pe = jnp.asarray(_pe)
    w_row = lin_W.reshape(1, D)
    lb_row = lin_b.reshape(1, D)
    gb_row = gcn_b.reshape(1, D)
    out = _run_tc2(x3, parts, xlin, dinv, pe, w_row, lb_row, gb_row)
    return out
